# baseline (speedup 1.0000x reference)
import functools

import jax
import jax.numpy as jnp
from jax import lax
from jax.experimental import pallas as pl
from jax.experimental.pallas import tpu as pltpu
from jax.experimental.pallas import tpu_sc as plsc

D_MODEL = 2048
BATCH = 4
SEQ = 8192
N_ROWS = BATCH * SEQ
NUM_CORES = 2
NUM_SUBCORES = 16
NW = NUM_CORES * NUM_SUBCORES
RPW = N_ROWS // NW
WPB = SEQ // RPW
K = 16
CHUNKS = RPW // K
NBUF = 3
LOOPS = (CHUNKS - 1) // NBUF


@functools.partial(
    pl.kernel,
    out_type=jax.ShapeDtypeStruct((BATCH, SEQ, 1, D_MODEL), jnp.float32),
    mesh=plsc.VectorSubcoreMesh(core_axis_name="c", subcore_axis_name="s"),
    scratch_types=[
        pltpu.VMEM((RPW,), jnp.int32),
        [pltpu.VMEM((K, 1, D_MODEL), jnp.float32)] * NBUF,
        [pltpu.SemaphoreType.DMA] * NBUF,
        [pltpu.SemaphoreType.DMA] * NBUF,
    ],
)
def _sc_gather(pos_hbm, pe_hbm, out_hbm, idx_v, bufs, gsems, wsems):
    wid = lax.axis_index("s") * NUM_CORES + lax.axis_index("c")
    b = wid // WPB
    s0 = (wid % WPB) * RPW
    pltpu.sync_copy(pos_hbm.at[pl.ds(wid * RPW, RPW)], idx_v)

    def start_gather(i, p):
        idx_chunk = idx_v.at[pl.ds(i * K, K)]
        pltpu.async_copy(pe_hbm.at[idx_chunk], bufs[p].at[:, 0, :], gsems[p])

    def wait_gather(i, p):
        idx_chunk = idx_v.at[pl.ds(i * K, K)]
        pltpu.make_async_copy(
            pe_hbm.at[idx_chunk], bufs[p].at[:, 0, :], gsems[p]).wait()

    def start_wb(i, p):
        pltpu.async_copy(bufs[p], out_hbm.at[b, pl.ds(s0 + i * K, K)], wsems[p])

    def drain_wb(p):
        pltpu.make_async_copy(bufs[p], out_hbm.at[b, pl.ds(s0, K)], wsems[p]).wait()

    start_gather(0, 0)
    start_gather(1, 1)

    def ring_body(j, carry):
        for p in range(NBUF):
            i = NBUF * j + p
            q = (p + 2) % NBUF
            wait_gather(i, p)
            if p == 0:
                @pl.when(j > 0)
                def _():
                    drain_wb(q)
            else:
                drain_wb(q)

            start_wb(i, p)

            @pl.when(i + 2 < CHUNKS)
            def _():
                start_gather(i + 2, q)
        return carry

    lax.fori_loop(0, LOOPS, ring_body, 0)

    last = CHUNKS - 1
    wait_gather(last, 0)
    start_wb(last, 0)
    drain_wb(2)
    drain_wb(0)


def kernel(pos, pe):
    return _sc_gather(pos.reshape(N_ROWS), pe)
